# baseline (speedup 1.0000x reference)
import jax
import jax.numpy as jnp
from jax import lax
from jax.experimental import pallas as pl
from jax.experimental.pallas import tpu as pltpu

_N = 4096
_I32_MIN = -2147483648
_I32_MAX = 2147483647


def _key_to_f32(s):
    b = jnp.where(s < 0, s ^ jnp.int32(0x7FFFFFFF), s)
    return lax.bitcast_convert_type(b, jnp.float32)


def _body(x_ref, o_ref):
    x = x_ref[...]
    b = lax.bitcast_convert_type(x, jnp.int32)
    keys = jnp.where(b < 0, b ^ jnp.int32(0x7FFFFFFF), b).T
    rows = keys.shape[1]

    qidx = lax.broadcasted_iota(jnp.int32, (7, 1), 0)
    kvec = 512 * (qidx + 1) - 1
    fr = 0.875 - 0.125 * qidx.astype(jnp.float32)
    lo = jnp.full((7, rows), _I32_MIN, jnp.int32)
    ones_v = jnp.ones((1, _N), jnp.float32)
    kthr = (kvec + 1).astype(jnp.float32)

    def body(i, lo):
        step = jnp.int32(1) << (jnp.int32(31) - i)
        t = lo + step
        cnt = jnp.concatenate(
            [
                lax.dot_general(
                    ones_v,
                    (keys < t[q : q + 1, :]).astype(jnp.float32),
                    (((1,), (0,)), ((), ())),
                    preferred_element_type=jnp.float32,
                )
                for q in range(7)
            ],
            axis=0,
        )
        return jnp.where(cnt >= kthr, lo, t)

    s0 = lax.fori_loop(0, 32, body, lo)

    cnt_le = jnp.concatenate(
        [
            lax.dot_general(
                ones_v,
                (keys <= s0[q : q + 1, :]).astype(jnp.float32),
                (((1,), (0,)), ((), ())),
                preferred_element_type=jnp.float32,
            )
            for q in range(7)
        ],
        axis=0,
    )
    nxt = jnp.concatenate(
        [
            jnp.min(
                jnp.where(keys > s0[q : q + 1, :], keys, jnp.int32(_I32_MAX)),
                axis=0,
                keepdims=True,
            )
            for q in range(7)
        ],
        axis=0,
    )
    s1 = jnp.where(cnt_le >= kthr + 1.0, s0, nxt)

    v0 = _key_to_f32(s0)
    v1 = _key_to_f32(s1)
    e = v0 * (1.0 - fr) + v1 * fr
    e1, e2, e3, e4, e5, e6, e7 = [e[q : q + 1, :] for q in range(7)]

    sa = e4
    sb = e6 - e2
    sg = (e6 + e2 - 2.0 * e4) / sb
    sk = (e7 - e5 + e3 - e1) / sb
    o_ref[...] = jnp.concatenate([sa, sb, sg, sk], axis=0)


@jax.jit
def kernel(x):
    n = x.shape[0]
    block_rows = 1024
    out = pl.pallas_call(
        _body,
        grid=(n // block_rows,),
        in_specs=[pl.BlockSpec((block_rows, _N), lambda i: (i, 0))],
        out_specs=pl.BlockSpec((4, block_rows), lambda i: (0, i)),
        out_shape=jax.ShapeDtypeStruct((4, n), x.dtype),
        compiler_params=pltpu.CompilerParams(
            dimension_semantics=("parallel",)
        ),
    )(x)
    return out.T


# speedup vs baseline: 1.0151x; 1.0151x over previous
import jax
import jax.numpy as jnp
from jax import lax
from jax.experimental import pallas as pl
from jax.experimental.pallas import tpu as pltpu

_N = 4096
_I32_MIN = -2147483648
_I32_MAX = 2147483647


def _key_to_f32(s):
    b = jnp.where(s < 0, s ^ jnp.int32(0x7FFFFFFF), s)
    return lax.bitcast_convert_type(b, jnp.float32)


def _body(x_ref, o_ref):
    x = x_ref[...]
    b = lax.bitcast_convert_type(x, jnp.int32)
    keys = jnp.where(b < 0, b ^ jnp.int32(0x7FFFFFFF), b).T
    rows = keys.shape[1]

    qidx = lax.broadcasted_iota(jnp.int32, (7, 1), 0)
    kvec = 512 * (qidx + 1) - 1
    fr = 0.875 - 0.125 * qidx.astype(jnp.float32)
    lo = jnp.full((7, rows), _I32_MIN, jnp.int32)
    ones_v = jnp.ones((1, _N), jnp.float32)
    kthr = (kvec + 1).astype(jnp.float32)

    def body(i, lo):
        step = jnp.int32(1) << (jnp.int32(31) - i)
        t = lo + step
        cnt = jnp.concatenate(
            [
                lax.dot_general(
                    ones_v,
                    (keys < t[q : q + 1, :]).astype(jnp.float32),
                    (((1,), (0,)), ((), ())),
                    preferred_element_type=jnp.float32,
                )
                for q in range(7)
            ],
            axis=0,
        )
        return jnp.where(cnt >= kthr, lo, t)

    s0 = lax.fori_loop(0, 32, body, lo)

    cnt_le = jnp.concatenate(
        [
            lax.dot_general(
                ones_v,
                (keys <= s0[q : q + 1, :]).astype(jnp.float32),
                (((1,), (0,)), ((), ())),
                preferred_element_type=jnp.float32,
            )
            for q in range(7)
        ],
        axis=0,
    )
    nxt = jnp.concatenate(
        [
            jnp.min(
                jnp.where(keys > s0[q : q + 1, :], keys, jnp.int32(_I32_MAX)),
                axis=0,
                keepdims=True,
            )
            for q in range(7)
        ],
        axis=0,
    )
    s1 = jnp.where(cnt_le >= kthr + 1.0, s0, nxt)

    v0 = _key_to_f32(s0)
    v1 = _key_to_f32(s1)
    e = v0 * (1.0 - fr) + v1 * fr
    e1, e2, e3, e4, e5, e6, e7 = [e[q : q + 1, :] for q in range(7)]

    sa = e4
    sb = e6 - e2
    sg = (e6 + e2 - 2.0 * e4) / sb
    sk = (e7 - e5 + e3 - e1) / sb
    o_ref[...] = jnp.concatenate([sa, sb, sg, sk], axis=0)


@jax.jit
def kernel(x):
    n = x.shape[0]
    block_rows = 512
    out = pl.pallas_call(
        _body,
        grid=(n // block_rows,),
        in_specs=[pl.BlockSpec((block_rows, _N), lambda i: (i, 0))],
        out_specs=pl.BlockSpec((4, block_rows), lambda i: (0, i)),
        out_shape=jax.ShapeDtypeStruct((4, n), x.dtype),
        compiler_params=pltpu.CompilerParams(
            dimension_semantics=("parallel",)
        ),
    )(x)
    return out.T
